# baseline (speedup 1.0000x reference)
import functools

import jax
import jax.numpy as jnp
from jax import lax
from jax.experimental import pallas as pl
from jax.experimental.pallas import tpu as pltpu
from jax.experimental.pallas import tpu_sc as plsc

B, N = 8, 65536
H, W = 512, 512
NC, NS, L = 2, 16, 16
NW = NC * NS
P = B * N
PPT = P // NW
CHUNK = 8192
K = PPT // CHUNK
GSZ = 8192
NSTREAM = CHUNK // GSZ
GROUPS = CHUNK // 128


def _mask_body(coords_hbm, terrain_hbm, out_hbm,
               cbuf0, cbuf1, ibuf0, ibuf1, gbuf0, gbuf1, obuf0, obuf1,
               csem0, csem1, gsem0, gsem1, osem0, osem1):
    c = lax.axis_index("c")
    s = lax.axis_index("s")
    wid = s * NC + c
    base = wid * PPT
    batch = wid // (NW // B)
    bofs = batch * (H * W)

    cbufs, ibufs = (cbuf0, cbuf1), (ibuf0, ibuf1)
    gbufs, obufs = (gbuf0, gbuf1), (obuf0, obuf1)
    csems, gsems, osems = (csem0, csem1), (gsem0, gsem1), (osem0, osem1)

    def coords_dma(k):
        cb = base + k * CHUNK
        return pltpu.make_async_copy(
            coords_hbm.at[pl.ds(cb * 4, CHUNK * 4)], cbufs[k % 2],
            csems[k % 2])

    def out_dma(k):
        cb = base + k * CHUNK
        return pltpu.make_async_copy(
            obufs[k % 2], out_hbm.at[pl.ds(cb, CHUNK)], osems[k % 2])

    def gather(k, j):
        return pltpu.make_async_copy(
            terrain_hbm.at[ibufs[k % 2].at[pl.ds(j * GSZ, GSZ)]],
            gbufs[k % 2].at[pl.ds(j * GSZ, GSZ)], gsems[k % 2])

    def compute_idx(k):
        cbuf, ibuf = cbufs[k % 2], ibufs[k % 2]

        def body(g, carry):
            for r in range(8):
                off = g * 512 + r * 16
                xv = cbuf[pl.ds(off, L)]
                yv = cbuf[pl.ds(off + 128, L)]
                xq = (xv * 511.0).astype(jnp.int32)
                yq = (yv * 511.0).astype(jnp.int32)
                pidx = (bofs + (yq >> 3) * 3072 + (yq << 7)
                        + (xq >> 7) * 896 + xq)
                ibuf[pl.ds(g * 128 + r * 16, L)] = pidx
            return carry

        lax.fori_loop(0, GROUPS, body, 0)

    def compare(k):
        cbuf, gbuf, obuf = cbufs[k % 2], gbufs[k % 2], obufs[k % 2]

        def body(g, carry):
            for r in range(8):
                zv = cbuf[pl.ds(g * 512 + 256 + r * 16, L)]
                tv = gbuf[pl.ds(g * 128 + r * 16, L)]
                obuf[pl.ds(g * 128 + r * 16, L)] = jnp.where(
                    zv <= tv, 1.0, 0.0)
            return carry

        lax.fori_loop(0, GROUPS, body, 0)

    coords_dma(0).start()
    coords_dma(1).start()
    coords_dma(0).wait()
    compute_idx(0)
    for j in range(NSTREAM):
        gather(0, j).start()

    for k in range(K):
        if k + 1 < K:
            coords_dma(k + 1).wait()
            compute_idx(k + 1)
        for j in range(NSTREAM):
            gather(k, j).wait()
        if k + 1 < K:
            for j in range(NSTREAM):
                gather(k + 1, j).start()
        if k >= 2:
            out_dma(k - 2).wait()
        compare(k)
        out_dma(k).start()
        if k + 2 < K:
            coords_dma(k + 2).start()

    out_dma(K - 2).wait()
    out_dma(K - 1).wait()


@jax.jit
def _launch(coords_phys, terrain_phys):
    mesh = plsc.VectorSubcoreMesh(core_axis_name="c", subcore_axis_name="s")
    kern = functools.partial(
        pl.kernel,
        mesh=mesh,
        out_type=jax.ShapeDtypeStruct((P,), jnp.float32),
        scratch_types=[
            pltpu.VMEM((CHUNK * 4,), jnp.float32),
            pltpu.VMEM((CHUNK * 4,), jnp.float32),
            pltpu.VMEM((CHUNK,), jnp.int32),
            pltpu.VMEM((CHUNK,), jnp.int32),
            pltpu.VMEM((CHUNK,), jnp.float32),
            pltpu.VMEM((CHUNK,), jnp.float32),
            pltpu.VMEM((CHUNK,), jnp.float32),
            pltpu.VMEM((CHUNK,), jnp.float32),
            pltpu.SemaphoreType.DMA,
            pltpu.SemaphoreType.DMA,
            pltpu.SemaphoreType.DMA,
            pltpu.SemaphoreType.DMA,
            pltpu.SemaphoreType.DMA,
            pltpu.SemaphoreType.DMA,
        ],
        compiler_params=pltpu.CompilerParams(needs_layout_passes=False),
    )(_mask_body)
    return kern(coords_phys, terrain_phys)


def kernel(coords, terrain_height):
    coords_phys = (
        coords.reshape(B, N // 128, 128, 4)
        .transpose(0, 1, 3, 2)
        .reshape(-1)
    )
    terrain_phys = (
        terrain_height.reshape(B, H // 8, 8, W // 128, 128)
        .transpose(0, 1, 3, 2, 4)
        .reshape(-1)
    )
    out = _launch(coords_phys, terrain_phys)
    return out.reshape(B, N, 1)


# speedup vs baseline: 1.0365x; 1.0365x over previous
import functools

import jax
import jax.numpy as jnp
from jax import lax
from jax.experimental import pallas as pl
from jax.experimental.pallas import tpu as pltpu
from jax.experimental.pallas import tpu_sc as plsc

B, N = 8, 65536
H, W = 512, 512
NC, NS, L = 2, 16, 16
NW = NC * NS
P = B * N
PPT = P // NW
CHUNK = 4096
K = PPT // CHUNK
GSZ = 4096
NSTREAM = CHUNK // GSZ
GROUPS = CHUNK // 128


def _mask_body(coords_hbm, terrain_hbm, out_hbm,
               cbuf0, cbuf1, ibuf0, ibuf1, gbuf0, gbuf1, obuf0, obuf1,
               csem0, csem1, gsem0, gsem1, osem0, osem1):
    c = lax.axis_index("c")
    s = lax.axis_index("s")
    wid = s * NC + c
    base = wid * PPT
    batch = wid // (NW // B)
    bofs = batch * (H * W)

    cbufs, ibufs = (cbuf0, cbuf1), (ibuf0, ibuf1)
    gbufs, obufs = (gbuf0, gbuf1), (obuf0, obuf1)
    csems, gsems, osems = (csem0, csem1), (gsem0, gsem1), (osem0, osem1)

    def coords_dma(k):
        cb = base + k * CHUNK
        return pltpu.make_async_copy(
            coords_hbm.at[pl.ds(cb * 4, CHUNK * 4)], cbufs[k % 2],
            csems[k % 2])

    def out_dma(k):
        cb = base + k * CHUNK
        return pltpu.make_async_copy(
            obufs[k % 2], out_hbm.at[pl.ds(cb, CHUNK)], osems[k % 2])

    def gather(k, j):
        return pltpu.make_async_copy(
            terrain_hbm.at[ibufs[k % 2].at[pl.ds(j * GSZ, GSZ)]],
            gbufs[k % 2].at[pl.ds(j * GSZ, GSZ)], gsems[k % 2])

    def compute_idx(k):
        cbuf, ibuf = cbufs[k % 2], ibufs[k % 2]

        def body(g, carry):
            for r in range(8):
                off = g * 512 + r * 16
                xv = cbuf[pl.ds(off, L)]
                yv = cbuf[pl.ds(off + 128, L)]
                xq = (xv * 511.0).astype(jnp.int32)
                yq = (yv * 511.0).astype(jnp.int32)
                pidx = (bofs + (yq >> 3) * 3072 + (yq << 7)
                        + (xq >> 7) * 896 + xq)
                ibuf[pl.ds(g * 128 + r * 16, L)] = pidx
            return carry

        lax.fori_loop(0, GROUPS, body, 0)

    def compare(k):
        cbuf, gbuf, obuf = cbufs[k % 2], gbufs[k % 2], obufs[k % 2]

        def body(g, carry):
            for r in range(8):
                zv = cbuf[pl.ds(g * 512 + 256 + r * 16, L)]
                tv = gbuf[pl.ds(g * 128 + r * 16, L)]
                obuf[pl.ds(g * 128 + r * 16, L)] = jnp.where(
                    zv <= tv, 1.0, 0.0)
            return carry

        lax.fori_loop(0, GROUPS, body, 0)

    coords_dma(0).start()
    coords_dma(1).start()
    coords_dma(0).wait()
    compute_idx(0)
    for j in range(NSTREAM):
        gather(0, j).start()

    for k in range(K):
        if k + 1 < K:
            coords_dma(k + 1).wait()
            compute_idx(k + 1)
        for j in range(NSTREAM):
            gather(k, j).wait()
        if k + 1 < K:
            for j in range(NSTREAM):
                gather(k + 1, j).start()
        if k >= 2:
            out_dma(k - 2).wait()
        compare(k)
        out_dma(k).start()
        if k + 2 < K:
            coords_dma(k + 2).start()

    out_dma(K - 2).wait()
    out_dma(K - 1).wait()


@jax.jit
def _launch(coords_phys, terrain_phys):
    mesh = plsc.VectorSubcoreMesh(core_axis_name="c", subcore_axis_name="s")
    kern = functools.partial(
        pl.kernel,
        mesh=mesh,
        out_type=jax.ShapeDtypeStruct((P,), jnp.float32),
        scratch_types=[
            pltpu.VMEM((CHUNK * 4,), jnp.float32),
            pltpu.VMEM((CHUNK * 4,), jnp.float32),
            pltpu.VMEM((CHUNK,), jnp.int32),
            pltpu.VMEM((CHUNK,), jnp.int32),
            pltpu.VMEM((CHUNK,), jnp.float32),
            pltpu.VMEM((CHUNK,), jnp.float32),
            pltpu.VMEM((CHUNK,), jnp.float32),
            pltpu.VMEM((CHUNK,), jnp.float32),
            pltpu.SemaphoreType.DMA,
            pltpu.SemaphoreType.DMA,
            pltpu.SemaphoreType.DMA,
            pltpu.SemaphoreType.DMA,
            pltpu.SemaphoreType.DMA,
            pltpu.SemaphoreType.DMA,
        ],
        compiler_params=pltpu.CompilerParams(needs_layout_passes=False),
    )(_mask_body)
    return kern(coords_phys, terrain_phys)


def kernel(coords, terrain_height):
    coords_phys = (
        coords.reshape(B, N // 128, 128, 4)
        .transpose(0, 1, 3, 2)
        .reshape(-1)
    )
    terrain_phys = (
        terrain_height.reshape(B, H // 8, 8, W // 128, 128)
        .transpose(0, 1, 3, 2, 4)
        .reshape(-1)
    )
    out = _launch(coords_phys, terrain_phys)
    return out.reshape(B, N, 1)


# speedup vs baseline: 1.0500x; 1.0131x over previous
import functools

import jax
import jax.numpy as jnp
from jax import lax
from jax.experimental import pallas as pl
from jax.experimental.pallas import tpu as pltpu
from jax.experimental.pallas import tpu_sc as plsc

B, N = 8, 65536
H, W = 512, 512
NC, NS, L = 2, 16, 16
NW = NC * NS
P = B * N
PPT = P // NW
CHUNK = 4096
K = PPT // CHUNK
GSZ = 4096
NSTREAM = CHUNK // GSZ
GROUPS = CHUNK // 128


def _mask_body(coords_hbm, terrain_hbm, out_hbm,
               cbuf0, cbuf1, cbuf2, ibuf0, ibuf1, gbuf0, gbuf1,
               obuf0, obuf1,
               csem0, csem1, gsem0, gsem1, osem0, osem1):
    c = lax.axis_index("c")
    s = lax.axis_index("s")
    wid = s * NC + c
    base = wid * PPT
    batch = wid // (NW // B)
    bofs = batch * (H * W)

    cbufs, ibufs = (cbuf0, cbuf1, cbuf2), (ibuf0, ibuf1)
    gbufs, obufs = (gbuf0, gbuf1), (obuf0, obuf1)
    csems, gsems, osems = (csem0, csem1), (gsem0, gsem1), (osem0, osem1)

    def coords_dma(k):
        cb = base + k * CHUNK
        return pltpu.make_async_copy(
            coords_hbm.at[pl.ds(cb * 4, CHUNK * 4)], cbufs[k % 3],
            csems[k % 2])

    def out_dma(k):
        cb = base + k * CHUNK
        return pltpu.make_async_copy(
            obufs[k % 2], out_hbm.at[pl.ds(cb, CHUNK)], osems[k % 2])

    def gather(k, j):
        return pltpu.make_async_copy(
            terrain_hbm.at[ibufs[k % 2].at[pl.ds(j * GSZ, GSZ)]],
            gbufs[k % 2].at[pl.ds(j * GSZ, GSZ)], gsems[k % 2])

    def compute_idx(k):
        cbuf, ibuf = cbufs[k % 3], ibufs[k % 2]

        def body(g, carry):
            for r in range(8):
                off = g * 512 + r * 16
                xv = cbuf[pl.ds(off, L)]
                yv = cbuf[pl.ds(off + 128, L)]
                xq = (xv * 511.0).astype(jnp.int32)
                yq = (yv * 511.0).astype(jnp.int32)
                pidx = (bofs + (yq >> 3) * 3072 + (yq << 7)
                        + (xq >> 7) * 896 + xq)
                ibuf[pl.ds(g * 128 + r * 16, L)] = pidx
            return carry

        lax.fori_loop(0, GROUPS, body, 0)

    def compare(k):
        cbuf, gbuf, obuf = cbufs[k % 3], gbufs[k % 2], obufs[k % 2]

        def body(g, carry):
            for r in range(8):
                zv = cbuf[pl.ds(g * 512 + 256 + r * 16, L)]
                tv = gbuf[pl.ds(g * 128 + r * 16, L)]
                obuf[pl.ds(g * 128 + r * 16, L)] = jnp.where(
                    zv <= tv, 1.0, 0.0)
            return carry

        lax.fori_loop(0, GROUPS, body, 0)

    coords_dma(0).start()
    coords_dma(0).wait()
    coords_dma(1).start()
    compute_idx(0)
    for j in range(NSTREAM):
        gather(0, j).start()

    for k in range(K):
        if k + 1 < K:
            coords_dma(k + 1).wait()
            if k + 2 < K:
                coords_dma(k + 2).start()
            compute_idx(k + 1)
        for j in range(NSTREAM):
            gather(k, j).wait()
        if k + 1 < K:
            for j in range(NSTREAM):
                gather(k + 1, j).start()
        if k >= 2:
            out_dma(k - 2).wait()
        compare(k)
        out_dma(k).start()

    out_dma(K - 2).wait()
    out_dma(K - 1).wait()


@jax.jit
def _launch(coords_phys, terrain_phys):
    mesh = plsc.VectorSubcoreMesh(core_axis_name="c", subcore_axis_name="s")
    kern = functools.partial(
        pl.kernel,
        mesh=mesh,
        out_type=jax.ShapeDtypeStruct((P,), jnp.float32),
        scratch_types=[
            pltpu.VMEM((CHUNK * 4,), jnp.float32),
            pltpu.VMEM((CHUNK * 4,), jnp.float32),
            pltpu.VMEM((CHUNK * 4,), jnp.float32),
            pltpu.VMEM((CHUNK,), jnp.int32),
            pltpu.VMEM((CHUNK,), jnp.int32),
            pltpu.VMEM((CHUNK,), jnp.float32),
            pltpu.VMEM((CHUNK,), jnp.float32),
            pltpu.VMEM((CHUNK,), jnp.float32),
            pltpu.VMEM((CHUNK,), jnp.float32),
            pltpu.SemaphoreType.DMA,
            pltpu.SemaphoreType.DMA,
            pltpu.SemaphoreType.DMA,
            pltpu.SemaphoreType.DMA,
            pltpu.SemaphoreType.DMA,
            pltpu.SemaphoreType.DMA,
        ],
        compiler_params=pltpu.CompilerParams(needs_layout_passes=False),
    )(_mask_body)
    return kern(coords_phys, terrain_phys)


def kernel(coords, terrain_height):
    coords_phys = (
        coords.reshape(B, N // 128, 128, 4)
        .transpose(0, 1, 3, 2)
        .reshape(-1)
    )
    terrain_phys = (
        terrain_height.reshape(B, H // 8, 8, W // 128, 128)
        .transpose(0, 1, 3, 2, 4)
        .reshape(-1)
    )
    out = _launch(coords_phys, terrain_phys)
    return out.reshape(B, N, 1)
